# baseline (speedup 1.0000x reference)
import functools

import jax
import jax.numpy as jnp
from jax import lax
from jax.experimental import pallas as pl
from jax.experimental.pallas import tpu as pltpu
from jax.experimental.pallas import tpu_sc as plsc

N_NODES = 10000
N_EDGES = 160000
N_GRAPHS = 128
DIM = 64
HEADS = 4
HD = HEADS * DIM
GROUPS = 10
UCOLS = 128
NPAD = 10240
PROWS = NPAD * HEADS // 128

NW = 32


@functools.cache
def _sc_mesh():
    return plsc.VectorSubcoreMesh(
        core_axis_name="c", subcore_axis_name="s",
        num_cores=2, num_subcores=16)


def _f32dot(a, b):
    return jnp.dot(a, b, preferred_element_type=jnp.float32)


def _pre_body(pad, x_ref, w1_ref, b1_ref, w2_ref, b2_ref, g_ref, be_ref,
              o_ref):
    h = _f32dot(x_ref[...], w1_ref[...]) + b1_ref[...]
    h = h * jax.nn.sigmoid(h)
    h = _f32dot(h, w2_ref[...]) + b2_ref[...]
    mu = jnp.mean(h, axis=-1, keepdims=True)
    var = jnp.mean((h - mu) ** 2, axis=-1, keepdims=True)
    h = g_ref[...] * (h - mu) * lax.rsqrt(var + 1e-5) + be_ref[...]
    h = jax.nn.softplus(h)
    if pad:
        h = jnp.concatenate([h, jnp.zeros_like(h)], axis=1)
    o_ref[...] = h


def _pre_mlp(x, p, blk, pad):
    rows, k = x.shape
    grid = rows // blk
    ocols = 2 * DIM if pad else DIM
    consts = [p['W1'], p['b1'].reshape(1, DIM), p['W2'],
              p['b2'].reshape(1, DIM), p['ln_g'].reshape(1, DIM),
              p['ln_b'].reshape(1, DIM)]
    cspecs = [pl.BlockSpec(c.shape, lambda i: (0, 0)) for c in consts]
    return pl.pallas_call(
        functools.partial(_pre_body, pad),
        grid=(grid,),
        in_specs=[pl.BlockSpec((blk, k), lambda i: (i, 0))] + cspecs,
        out_specs=pl.BlockSpec((blk, ocols), lambda i: (i, 0)),
        out_shape=jax.ShapeDtypeStruct((rows, ocols), jnp.float32),
    )(x, *consts)


def _edge_body(e_ref, xi_ref, xj_ref, idx_ref, wt_ref, wb_ref, a1_ref,
               a2_ref, bns_ref, bno_ref, msk_ref, r_ref,
               ua_ref, ub_ref, p_ref, prow_ref):
    ew = _f32dot(e_ref[...], wb_ref[...])
    ci = jax.nn.softplus(_f32dot(xi_ref[...], wt_ref[...]) + ew)
    cj = jax.nn.softplus(_f32dot(xj_ref[...], wt_ref[...]) + ew)
    ar = _f32dot(ci, a1_ref[...]) + _f32dot(cj, a2_ref[...])
    al = jax.nn.softplus(bns_ref[...] * jax.nn.softplus(ar) + bno_ref[...])
    p16 = jnp.exp(al) * msk_ref[...]
    u = cj * _f32dot(p16, r_ref[...])
    ua_ref[...] = u[:, :128]
    ub_ref[...] = u[:, 128:]
    nid = idx_ref[...]
    base = (nid & 31) * 4
    ci32 = lax.broadcasted_iota(jnp.int32, p_ref.shape, 1)
    p128 = jnp.zeros(p_ref.shape, jnp.float32)
    for h in range(HEADS):
        p128 = jnp.where(ci32 == base + h, p16[:, h:h + 1], p128)
    p_ref[...] = p128
    prow_ref[...] = lax.shift_right_logical(nid, 5)


def _edge_kernel(e, xij, idx2d, consts, blk):
    grid = N_EDGES // blk
    njump = N_EDGES // blk
    cspecs = [pl.BlockSpec(c.shape, lambda i: (0, 0)) for c in consts]
    ospec = pl.BlockSpec((blk, UCOLS), lambda i: (i, 0))
    oshape = jax.ShapeDtypeStruct((N_EDGES, UCOLS), jnp.float32)
    return pl.pallas_call(
        _edge_body,
        grid=(grid,),
        in_specs=[pl.BlockSpec((blk, DIM), lambda i: (i, 0)),
                  pl.BlockSpec((blk, 2 * DIM), lambda i: (i, 0)),
                  pl.BlockSpec((blk, 2 * DIM), lambda i: (i + njump, 0)),
                  pl.BlockSpec((blk, 1), lambda i: (i, 0))] + cspecs,
        out_specs=[ospec, ospec, ospec,
                   pl.BlockSpec((blk, 1), lambda i: (i, 0))],
        out_shape=[oshape, oshape, oshape,
                   jax.ShapeDtypeStruct((N_EDGES, 1), jnp.int32)],
    )(e, xij, xij, idx2d, *consts)


def _head_mean(ua_ref, ub_ref, s_ref, s2_ref, bias_ref):
    ua = ua_ref[...]
    ub = ub_ref[...]
    s = s_ref[...] + s2_ref[...] + 1e-16
    h = (ua[:, 0:64] / s[:, 0:1] + ua[:, 64:128] / s[:, 1:2]
         + ub[:, 0:64] / s[:, 2:3] + ub[:, 64:128] / s[:, 3:4])
    return h * 0.25 + bias_ref[...]


def _node_body(ua_ref, ub_ref, s_ref, s2_ref, bias_ref, lin_ref, sc_ref,
               off_ref, o_ref):
    h = _head_mean(ua_ref, ub_ref, s_ref, s2_ref, bias_ref)
    sm = jax.nn.softmax(_f32dot(h, lin_ref[...]), axis=-1)
    h = h + 0.01 * (h * _f32dot(sm, sc_ref[...]) + off_ref[...])
    o_ref[...] = jnp.concatenate([h, jnp.zeros_like(h)], axis=1)


def _node_last_body(ua_ref, ub_ref, s_ref, s2_ref, h0_ref, bvec_ref,
                    bias_ref, lin_ref, sc_ref, off_ref, pool_ref):
    h = _head_mean(ua_ref, ub_ref, s_ref, s2_ref, bias_ref)
    sm = jax.nn.softmax(_f32dot(h, lin_ref[...]), axis=-1)
    h = h + 0.01 * (h * _f32dot(sm, sc_ref[...]) + off_ref[...])
    h = h + h0_ref[...][:, :DIM]
    gid = lax.broadcasted_iota(jnp.int32, (h.shape[0], N_GRAPHS), 1)
    onehot = (bvec_ref[...] == gid).astype(jnp.float32)
    part = lax.dot_general(onehot, h, (((0,), (0,)), ((), ())),
                           preferred_element_type=jnp.float32)

    @pl.when(pl.program_id(0) == 0)
    def _init():
        pool_ref[...] = jnp.zeros_like(pool_ref)

    pool_ref[...] += part


def _node_kernel(ua, ub, sarr, sarr2, consts, blk):
    grid = N_NODES // blk
    cspecs = [pl.BlockSpec(c.shape, lambda i: (0, 0)) for c in consts]
    return pl.pallas_call(
        _node_body,
        grid=(grid,),
        in_specs=[pl.BlockSpec((blk, UCOLS), lambda i: (i, 0)),
                  pl.BlockSpec((blk, UCOLS), lambda i: (i, 0)),
                  pl.BlockSpec((blk, HEADS), lambda i: (i, 0)),
                  pl.BlockSpec((blk, HEADS), lambda i: (i, 0))] + cspecs,
        out_specs=pl.BlockSpec((blk, 2 * DIM), lambda i: (i, 0)),
        out_shape=jax.ShapeDtypeStruct((N_NODES, 2 * DIM), jnp.float32),
    )(ua, ub, sarr, sarr2, *consts)


def _node_last_kernel(ua, ub, sarr, sarr2, h0, bvec, consts, blk):
    grid = N_NODES // blk
    cspecs = [pl.BlockSpec(c.shape, lambda i: (0, 0)) for c in consts]
    return pl.pallas_call(
        _node_last_body,
        grid=(grid,),
        in_specs=[pl.BlockSpec((blk, UCOLS), lambda i: (i, 0)),
                  pl.BlockSpec((blk, UCOLS), lambda i: (i, 0)),
                  pl.BlockSpec((blk, HEADS), lambda i: (i, 0)),
                  pl.BlockSpec((blk, HEADS), lambda i: (i, 0)),
                  pl.BlockSpec((blk, 2 * DIM), lambda i: (i, 0)),
                  pl.BlockSpec((blk, 1), lambda i: (i, 0))] + cspecs,
        out_specs=pl.BlockSpec((N_GRAPHS, DIM), lambda i: (0, 0)),
        out_shape=jax.ShapeDtypeStruct((N_GRAPHS, DIM), jnp.float32),
    )(ua, ub, sarr, sarr2, h0, bvec, *consts)


def _final_body(pw_ref, pm_ref, w_ref, b_ref, o_ref):
    d = pm_ref[...] - pw_ref[...]
    o_ref[...] = _f32dot(d, w_ref[...]) + b_ref[...]


def _final(pool_w, pool_m, w, b):
    return pl.pallas_call(
        _final_body,
        in_specs=[pl.BlockSpec((N_GRAPHS, DIM), lambda: (0, 0)),
                  pl.BlockSpec((N_GRAPHS, DIM), lambda: (0, 0)),
                  pl.BlockSpec((DIM, 1), lambda: (0, 0)),
                  pl.BlockSpec((1, 1), lambda: (0, 0))],
        out_specs=pl.BlockSpec((N_GRAPHS, 1), lambda: (0, 0)),
        out_shape=jax.ShapeDtypeStruct((N_GRAPHS, 1), jnp.float32),
    )(pool_w, pool_m, w, b.reshape(1, 1))


def _gather_rows(table, idx):
    etot = idx.shape[0]
    cols = table.shape[1]
    per_w = etot // NW
    ch = 80
    n_ch = per_w // ch

    nb = 5
    n_grp = n_ch // nb

    @functools.partial(
        pl.kernel,
        out_type=jax.ShapeDtypeStruct((etot, cols), jnp.float32),
        mesh=_sc_mesh(),
        scratch_types=[pltpu.VMEM((per_w,), jnp.int32)]
        + [pltpu.VMEM((ch, cols), jnp.float32)] * nb
        + [pltpu.SemaphoreType.DMA, pltpu.SemaphoreType.DMA],
    )
    def k(table_hbm, idx_hbm, out_hbm, idx_v, *rest):
        bufs = rest[:nb]
        semg, semw = rest[nb:]
        wid = lax.axis_index("s") * 2 + lax.axis_index("c")
        base = wid * per_w
        pltpu.sync_copy(idx_hbm.at[pl.ds(base, per_w)], idx_v)

        def body(g, _):
            offs = [(g * nb + j) * ch for j in range(nb)]

            @pl.when(g > 0)
            def _():
                for j in range(nb):
                    pltpu.make_async_copy(
                        bufs[j], out_hbm.at[pl.ds(base, ch)], semw).wait()

            gds = [pltpu.async_copy(
                table_hbm.at[idx_v.at[pl.ds(offs[j], ch)]], bufs[j], semg)
                for j in range(nb)]
            for d in gds:
                d.wait()
            for j in range(nb):
                pltpu.async_copy(
                    bufs[j], out_hbm.at[pl.ds(base + offs[j], ch)], semw)
            return ()

        lax.fori_loop(0, n_grp, body, (), unroll=False)
        for j in range(nb):
            pltpu.make_async_copy(
                bufs[j], out_hbm.at[pl.ds(base, ch)], semw).wait()

    return k(table, idx)


def _scatter_add(ua, ub, p128, prow, idx, zeros):
    etot = idx.shape[0]
    per_t = etot // 16
    ch = 40
    n_ch = per_t // ch
    rpt = NPAD // 16
    prt = PROWS // 10

    oshape = jax.ShapeDtypeStruct((NPAD, UCOLS), jnp.float32)
    pshape = jax.ShapeDtypeStruct((PROWS, 128), jnp.float32)

    nb = 2
    n_grp = n_ch // nb

    @functools.partial(
        pl.kernel,
        out_type=[oshape, oshape, pshape, pshape],
        mesh=_sc_mesh(),
        scratch_types=[pltpu.VMEM((ch,), jnp.int32)] * nb
        + [pltpu.VMEM((ch,), jnp.int32)] * nb
        + [pltpu.VMEM((ch, UCOLS), jnp.float32)] * nb
        + [pltpu.VMEM((ch, 128), jnp.float32)] * nb
        + [pltpu.VMEM_SHARED((NPAD, UCOLS), jnp.float32),
           pltpu.VMEM_SHARED((PROWS, 128), jnp.float32),
           pltpu.SemaphoreType.DMA, pltpu.SemaphoreType.DMA],
    )
    def k(ua_hbm, ub_hbm, p_hbm, prow_hbm, idx_hbm, z_hbm,
          oa_hbm, ob_hbm, op_hbm, op2_hbm, *rest):
        idxb = rest[0:nb]
        prowb = rest[nb:2 * nb]
        rowsb = rest[2 * nb:3 * nb]
        pb = rest[3 * nb:4 * nb]
        acc, accp, sems, sema = rest[4 * nb:]
        cid = lax.axis_index("c")
        sid = lax.axis_index("s")
        pltpu.sync_copy(z_hbm.at[pl.ds(sid * rpt, rpt)],
                        acc.at[pl.ds(sid * rpt, rpt)])

        @pl.when(sid < 10)
        def _():
            pltpu.sync_copy(z_hbm.at[pl.ds(sid * prt, prt)],
                            accp.at[pl.ds(sid * prt, prt)])

        plsc.subcore_barrier()

        def body(g, _):
            offs = [sid * per_t + (g * nb + j) * ch for j in range(nb)]
            ids = [pltpu.async_copy(idx_hbm.at[pl.ds(offs[j], ch)],
                                    idxb[j], sems) for j in range(nb)]

            for d in ids:
                d.wait()

            def stage_and_add(rows_hbm, mine):
                ds = [pltpu.async_copy(
                    rows_hbm.at[pl.ds(offs[j], ch)], rowsb[j], sems)
                    for j in range(nb)]
                for j in mine:
                    ds.append(pltpu.async_copy(
                        prow_hbm.at[pl.ds(offs[j], ch)], prowb[j], sems))
                    ds.append(pltpu.async_copy(
                        p_hbm.at[pl.ds(offs[j], ch)], pb[j], sems))
                for d in ds:
                    d.wait()
                ads = [pltpu.async_copy(rowsb[j], acc.at[idxb[j]], sema,
                                        add=True) for j in range(nb)]
                ads += [pltpu.async_copy(pb[j], accp.at[prowb[j]], sema,
                                         add=True) for j in mine]
                for d in ads:
                    d.wait()

            @pl.when(cid == 0)
            def _():
                stage_and_add(ua_hbm, [0])

            @pl.when(cid == 1)
            def _():
                stage_and_add(ub_hbm, [1])

            return ()

        lax.fori_loop(0, n_grp, body, (), unroll=False)
        plsc.subcore_barrier()

        @pl.when(cid == 0)
        def _():
            pltpu.sync_copy(acc.at[pl.ds(sid * rpt, rpt)],
                            oa_hbm.at[pl.ds(sid * rpt, rpt)])

        @pl.when((cid == 0) & (sid < 10))
        def _():
            pltpu.sync_copy(accp.at[pl.ds(sid * prt, prt)],
                            op_hbm.at[pl.ds(sid * prt, prt)])

        @pl.when(cid == 1)
        def _():
            pltpu.sync_copy(acc.at[pl.ds(sid * rpt, rpt)],
                            ob_hbm.at[pl.ds(sid * rpt, rpt)])

        @pl.when((cid == 1) & (sid < 10))
        def _():
            pltpu.sync_copy(accp.at[pl.ds(sid * prt, prt)],
                            op2_hbm.at[pl.ds(sid * prt, prt)])

    return k(ua, ub, p128, prow, idx, zeros)


def _layer_consts(cp):
    w = cp['W']
    w_top = jnp.concatenate([w[:DIM], jnp.zeros((DIM, HD), jnp.float32)], 0)
    w_bot = w[DIM:]
    att = cp['att'][0]
    a1 = jnp.zeros((HD, 16), jnp.float32)
    a2 = jnp.zeros((HD, 16), jnp.float32)
    for h in range(HEADS):
        a1 = a1.at[h * DIM:(h + 1) * DIM, h].set(att[h, :DIM])
        a2 = a2.at[h * DIM:(h + 1) * DIM, h].set(att[h, DIM:])
    scale = cp['bn_g'] * lax.rsqrt(cp['bn_v'] + 1e-5)
    offs = cp['bn_b'] - cp['bn_m'] * scale
    bns = jnp.zeros((1, 16), jnp.float32).at[0, :HEADS].set(scale)
    bno = jnp.zeros((1, 16), jnp.float32).at[0, :HEADS].set(offs)
    msk = jnp.zeros((1, 16), jnp.float32).at[0, :HEADS].set(1.0)
    r16 = jnp.zeros((16, HD), jnp.float32)
    for h in range(HEADS):
        r16 = r16.at[h, h * DIM:(h + 1) * DIM].set(1.0)
    return [w_top, w_bot, a1, a2, bns, bno, msk, r16]


def _dgn_consts(dp):
    scale = (dp['bn_g'] * lax.rsqrt(dp['bn_v'] + 1e-5)).reshape(GROUPS, DIM)
    offs = (dp['bn_b'].reshape(GROUPS, DIM)
            - dp['bn_m'].reshape(GROUPS, DIM) * scale)
    return dp['lin'], scale, offs.sum(0).reshape(1, DIM)


def _branch(x, edge_index, edge_attr, batch, params, zeros_acc):
    idx_i = edge_index[0].astype(jnp.int32)
    idx_j = edge_index[1].astype(jnp.int32)
    idx_i2d = idx_i.reshape(N_EDGES, 1)
    bvec = batch.astype(jnp.int32).reshape(N_NODES, 1)

    idx_ij = jnp.concatenate([idx_i, idx_j])

    h0 = _pre_mlp(x[:, 1:21], params['preN'], 1000, pad=True)
    e = _pre_mlp(edge_attr, params['preE'], 1000, pad=False)

    h = h0
    for l in range(3):
        econsts = _layer_consts(params['convs'][l])
        lin, sc, off = _dgn_consts(params['dgns'][l])
        nconsts = [params['convs'][l]['bias'].reshape(1, DIM), lin, sc, off]
        xij = _gather_rows(h, idx_ij)
        ua, ub, p128, prow = _edge_kernel(e, xij, idx_i2d, econsts, 1000)
        acca, accb, accp, accp2 = _scatter_add(ua, ub, p128,
                                               prow.reshape(-1),
                                               idx_i, zeros_acc)
        sarr = accp.reshape(NPAD, HEADS)
        sarr2 = accp2.reshape(NPAD, HEADS)
        if l < 2:
            h = _node_kernel(acca, accb, sarr, sarr2, nconsts, 1000)
        else:
            pool = _node_last_kernel(acca, accb, sarr, sarr2, h0, bvec,
                                     nconsts, 1000)
    return pool


@jax.jit
def _forward_impl(wild_x, wild_edge_index, wild_edge_attr, wild_batch,
                  mutant_x, mutant_edge_index, mutant_edge_attr, mutant_batch,
                  params):
    zeros_acc = jnp.zeros((NPAD, UCOLS), jnp.float32)
    pw = _branch(wild_x, wild_edge_index, wild_edge_attr, wild_batch,
                 params, zeros_acc)
    pm = _branch(mutant_x, mutant_edge_index, mutant_edge_attr, mutant_batch,
                 params, zeros_acc)
    out = _final(pw, pm, params['out_W'], params['out_b'])
    return out.reshape(-1)


def kernel(wild_x, wild_edge_index, wild_edge_attr, wild_batch,
           mutant_x, mutant_edge_index, mutant_edge_attr, mutant_batch,
           params):
    return _forward_impl(wild_x, wild_edge_index, wild_edge_attr, wild_batch,
                         mutant_x, mutant_edge_index, mutant_edge_attr,
                         mutant_batch, params)


# speedup vs baseline: 1.0774x; 1.0774x over previous
import functools

import jax
import jax.numpy as jnp
from jax import lax
from jax.experimental import pallas as pl
from jax.experimental.pallas import tpu as pltpu
from jax.experimental.pallas import tpu_sc as plsc

N_NODES = 10000
N_EDGES = 160000
N_GRAPHS = 128
DIM = 64
HEADS = 4
HD = HEADS * DIM
GROUPS = 10
UCOLS = 128
NPAD = 10240
PROWS = NPAD * HEADS // 128

NW = 32


@functools.cache
def _sc_mesh():
    return plsc.VectorSubcoreMesh(
        core_axis_name="c", subcore_axis_name="s",
        num_cores=2, num_subcores=16)


def _f32dot(a, b):
    return jnp.dot(a, b, preferred_element_type=jnp.float32)


def _pre_body(pad, x_ref, w1_ref, b1_ref, w2_ref, b2_ref, g_ref, be_ref,
              o_ref):
    h = _f32dot(x_ref[...], w1_ref[...]) + b1_ref[...]
    h = h * jax.nn.sigmoid(h)
    h = _f32dot(h, w2_ref[...]) + b2_ref[...]
    mu = jnp.mean(h, axis=-1, keepdims=True)
    var = jnp.mean((h - mu) ** 2, axis=-1, keepdims=True)
    h = g_ref[...] * (h - mu) * lax.rsqrt(var + 1e-5) + be_ref[...]
    h = jax.nn.softplus(h)
    if pad:
        h = jnp.concatenate([h, jnp.zeros_like(h)], axis=1)
    o_ref[...] = h


def _pre_mlp(x, p, blk, pad):
    rows, k = x.shape
    grid = rows // blk
    ocols = 2 * DIM if pad else DIM
    consts = [p['W1'], p['b1'].reshape(1, DIM), p['W2'],
              p['b2'].reshape(1, DIM), p['ln_g'].reshape(1, DIM),
              p['ln_b'].reshape(1, DIM)]
    cspecs = [pl.BlockSpec(c.shape, lambda i: (0, 0)) for c in consts]
    return pl.pallas_call(
        functools.partial(_pre_body, pad),
        grid=(grid,),
        in_specs=[pl.BlockSpec((blk, k), lambda i: (i, 0))] + cspecs,
        out_specs=pl.BlockSpec((blk, ocols), lambda i: (i, 0)),
        out_shape=jax.ShapeDtypeStruct((rows, ocols), jnp.float32),
    )(x, *consts)


def _edge_body(e_ref, xi_ref, xj_ref, idx_ref, wt_ref, wb_ref, a1_ref,
               a2_ref, bns_ref, bno_ref, msk_ref, r_ref,
               ua_ref, ub_ref, p_ref, prow_ref):
    ew = _f32dot(e_ref[...], wb_ref[...])
    ci = jax.nn.softplus(_f32dot(xi_ref[...], wt_ref[...]) + ew)
    cj = jax.nn.softplus(_f32dot(xj_ref[...], wt_ref[...]) + ew)
    ar = _f32dot(ci, a1_ref[...]) + _f32dot(cj, a2_ref[...])
    al = jax.nn.softplus(bns_ref[...] * jax.nn.softplus(ar) + bno_ref[...])
    p16 = jnp.exp(al) * msk_ref[...]
    u = cj * _f32dot(p16, r_ref[...])
    ua_ref[...] = u[:, :128]
    ub_ref[...] = u[:, 128:]
    nid = idx_ref[...]
    base = (nid & 31) * 4
    ci32 = lax.broadcasted_iota(jnp.int32, p_ref.shape, 1)
    p128 = jnp.zeros(p_ref.shape, jnp.float32)
    for h in range(HEADS):
        p128 = jnp.where(ci32 == base + h, p16[:, h:h + 1], p128)
    p_ref[...] = p128
    prow_ref[...] = lax.shift_right_logical(nid, 5)


def _edge_kernel(e, xij, idx2d, consts, blk):
    grid = N_EDGES // blk
    njump = N_EDGES // blk
    cspecs = [pl.BlockSpec(c.shape, lambda i: (0, 0)) for c in consts]
    ospec = pl.BlockSpec((blk, UCOLS), lambda i: (i, 0))
    oshape = jax.ShapeDtypeStruct((N_EDGES, UCOLS), jnp.float32)
    return pl.pallas_call(
        _edge_body,
        grid=(grid,),
        in_specs=[pl.BlockSpec((blk, DIM), lambda i: (i, 0)),
                  pl.BlockSpec((blk, 2 * DIM), lambda i: (i, 0)),
                  pl.BlockSpec((blk, 2 * DIM), lambda i: (i + njump, 0)),
                  pl.BlockSpec((blk, 1), lambda i: (i, 0))] + cspecs,
        out_specs=[ospec, ospec, ospec,
                   pl.BlockSpec((blk, 1), lambda i: (i, 0))],
        out_shape=[oshape, oshape, oshape,
                   jax.ShapeDtypeStruct((N_EDGES, 1), jnp.int32)],
    )(e, xij, xij, idx2d, *consts)


def _head_mean(ua_ref, ub_ref, s_ref, s2_ref, bias_ref):
    ua = ua_ref[...]
    ub = ub_ref[...]
    s = s_ref[...] + s2_ref[...] + 1e-16
    h = (ua[:, 0:64] / s[:, 0:1] + ua[:, 64:128] / s[:, 1:2]
         + ub[:, 0:64] / s[:, 2:3] + ub[:, 64:128] / s[:, 3:4])
    return h * 0.25 + bias_ref[...]


def _node_body(ua_ref, ub_ref, s_ref, s2_ref, bias_ref, lin_ref, sc_ref,
               off_ref, o_ref):
    h = _head_mean(ua_ref, ub_ref, s_ref, s2_ref, bias_ref)
    sm = jax.nn.softmax(_f32dot(h, lin_ref[...]), axis=-1)
    h = h + 0.01 * (h * _f32dot(sm, sc_ref[...]) + off_ref[...])
    o_ref[...] = jnp.concatenate([h, jnp.zeros_like(h)], axis=1)


def _node_last_body(ua_ref, ub_ref, s_ref, s2_ref, h0_ref, bvec_ref,
                    bias_ref, lin_ref, sc_ref, off_ref, pool_ref):
    h = _head_mean(ua_ref, ub_ref, s_ref, s2_ref, bias_ref)
    sm = jax.nn.softmax(_f32dot(h, lin_ref[...]), axis=-1)
    h = h + 0.01 * (h * _f32dot(sm, sc_ref[...]) + off_ref[...])
    h = h + h0_ref[...][:, :DIM]
    gid = lax.broadcasted_iota(jnp.int32, (h.shape[0], N_GRAPHS), 1)
    onehot = (bvec_ref[...] == gid).astype(jnp.float32)
    part = lax.dot_general(onehot, h, (((0,), (0,)), ((), ())),
                           preferred_element_type=jnp.float32)

    @pl.when(pl.program_id(0) == 0)
    def _init():
        pool_ref[...] = jnp.zeros_like(pool_ref)

    pool_ref[...] += part


def _node_kernel(ua, ub, sarr, sarr2, consts, blk):
    grid = N_NODES // blk
    cspecs = [pl.BlockSpec(c.shape, lambda i: (0, 0)) for c in consts]
    return pl.pallas_call(
        _node_body,
        grid=(grid,),
        in_specs=[pl.BlockSpec((blk, UCOLS), lambda i: (i, 0)),
                  pl.BlockSpec((blk, UCOLS), lambda i: (i, 0)),
                  pl.BlockSpec((blk, HEADS), lambda i: (i, 0)),
                  pl.BlockSpec((blk, HEADS), lambda i: (i, 0))] + cspecs,
        out_specs=pl.BlockSpec((blk, 2 * DIM), lambda i: (i, 0)),
        out_shape=jax.ShapeDtypeStruct((N_NODES, 2 * DIM), jnp.float32),
    )(ua, ub, sarr, sarr2, *consts)


def _node_last_kernel(ua, ub, sarr, sarr2, h0, bvec, consts, blk):
    grid = N_NODES // blk
    cspecs = [pl.BlockSpec(c.shape, lambda i: (0, 0)) for c in consts]
    return pl.pallas_call(
        _node_last_body,
        grid=(grid,),
        in_specs=[pl.BlockSpec((blk, UCOLS), lambda i: (i, 0)),
                  pl.BlockSpec((blk, UCOLS), lambda i: (i, 0)),
                  pl.BlockSpec((blk, HEADS), lambda i: (i, 0)),
                  pl.BlockSpec((blk, HEADS), lambda i: (i, 0)),
                  pl.BlockSpec((blk, 2 * DIM), lambda i: (i, 0)),
                  pl.BlockSpec((blk, 1), lambda i: (i, 0))] + cspecs,
        out_specs=pl.BlockSpec((N_GRAPHS, DIM), lambda i: (0, 0)),
        out_shape=jax.ShapeDtypeStruct((N_GRAPHS, DIM), jnp.float32),
    )(ua, ub, sarr, sarr2, h0, bvec, *consts)


def _final_body(pw_ref, pm_ref, w_ref, b_ref, o_ref):
    d = pm_ref[...] - pw_ref[...]
    o_ref[...] = _f32dot(d, w_ref[...]) + b_ref[...]


def _final(pool_w, pool_m, w, b):
    return pl.pallas_call(
        _final_body,
        in_specs=[pl.BlockSpec((N_GRAPHS, DIM), lambda: (0, 0)),
                  pl.BlockSpec((N_GRAPHS, DIM), lambda: (0, 0)),
                  pl.BlockSpec((DIM, 1), lambda: (0, 0)),
                  pl.BlockSpec((1, 1), lambda: (0, 0))],
        out_specs=pl.BlockSpec((N_GRAPHS, 1), lambda: (0, 0)),
        out_shape=jax.ShapeDtypeStruct((N_GRAPHS, 1), jnp.float32),
    )(pool_w, pool_m, w, b.reshape(1, 1))


def _gather_rows(table, idx):
    etot = idx.shape[0]
    cols = table.shape[1]
    per_w = etot // NW
    ch = 80
    n_ch = per_w // ch

    nb = 5
    n_grp = n_ch // nb

    @functools.partial(
        pl.kernel,
        out_type=jax.ShapeDtypeStruct((etot, cols), jnp.float32),
        mesh=_sc_mesh(),
        scratch_types=[pltpu.VMEM((per_w,), jnp.int32)]
        + [pltpu.VMEM((ch, cols), jnp.float32)] * nb
        + [pltpu.SemaphoreType.DMA, pltpu.SemaphoreType.DMA],
    )
    def k(table_hbm, idx_hbm, out_hbm, idx_v, *rest):
        bufs = rest[:nb]
        semg, semw = rest[nb:]
        wid = lax.axis_index("s") * 2 + lax.axis_index("c")
        base = wid * per_w
        pltpu.sync_copy(idx_hbm.at[pl.ds(base, per_w)], idx_v)

        def body(g, _):
            offs = [(g * nb + j) * ch for j in range(nb)]

            @pl.when(g > 0)
            def _():
                for j in range(nb):
                    pltpu.make_async_copy(
                        bufs[j], out_hbm.at[pl.ds(base, ch)], semw).wait()

            gds = [pltpu.async_copy(
                table_hbm.at[idx_v.at[pl.ds(offs[j], ch)]], bufs[j], semg)
                for j in range(nb)]
            for d in gds:
                d.wait()
            for j in range(nb):
                pltpu.async_copy(
                    bufs[j], out_hbm.at[pl.ds(base + offs[j], ch)], semw)
            return ()

        lax.fori_loop(0, n_grp, body, (), unroll=False)
        for j in range(nb):
            pltpu.make_async_copy(
                bufs[j], out_hbm.at[pl.ds(base, ch)], semw).wait()

    return k(table, idx)


def _scatter_add(ua, ub, p128, prow, idx, zeros):
    etot = idx.shape[0]
    per_t = etot // 16
    ch = 40
    n_ch = per_t // ch
    rpt = NPAD // 16
    prt = PROWS // 10

    oshape = jax.ShapeDtypeStruct((NPAD, UCOLS), jnp.float32)
    pshape = jax.ShapeDtypeStruct((PROWS, 128), jnp.float32)

    nb = 2
    nbuf = 2 * nb

    @functools.partial(
        pl.kernel,
        out_type=[oshape, oshape, pshape, pshape],
        mesh=_sc_mesh(),
        scratch_types=[pltpu.VMEM((ch,), jnp.int32)] * nbuf
        + [pltpu.VMEM((ch,), jnp.int32)] * nbuf
        + [pltpu.VMEM((ch, UCOLS), jnp.float32)] * nbuf
        + [pltpu.VMEM((ch, 128), jnp.float32)] * nbuf
        + [pltpu.VMEM_SHARED((NPAD, UCOLS), jnp.float32),
           pltpu.VMEM_SHARED((PROWS, 128), jnp.float32),
           pltpu.SemaphoreType.DMA, pltpu.SemaphoreType.DMA],
    )
    def k(ua_hbm, ub_hbm, p_hbm, prow_hbm, idx_hbm, z_hbm,
          oa_hbm, ob_hbm, op_hbm, op2_hbm, *rest):
        idxb = rest[0:nbuf]
        prowb = rest[nbuf:2 * nbuf]
        rowsb = rest[2 * nbuf:3 * nbuf]
        pb = rest[3 * nbuf:4 * nbuf]
        acc, accp, sems, sema = rest[4 * nbuf:]
        cid = lax.axis_index("c")
        sid = lax.axis_index("s")
        pltpu.sync_copy(z_hbm.at[pl.ds(sid * rpt, rpt)],
                        acc.at[pl.ds(sid * rpt, rpt)])

        @pl.when(sid < 10)
        def _():
            pltpu.sync_copy(z_hbm.at[pl.ds(sid * prt, prt)],
                            accp.at[pl.ds(sid * prt, prt)])

        plsc.subcore_barrier()

        def drain_adds(bs):
            pltpu.make_async_copy(rowsb[bs], acc.at[pl.ds(0, ch)],
                                  sema).wait()
            pltpu.make_async_copy(rowsb[bs + 1], acc.at[pl.ds(0, ch)],
                                  sema).wait()
            pltpu.make_async_copy(pb[bs], accp.at[pl.ds(0, ch)],
                                  sema).wait()

        def group(bs, c0, first):
            js = [bs, bs + 1]
            offs = [sid * per_t + (c0 + i) * ch for i in range(2)]

            @pl.when(jnp.logical_not(first))
            def _():
                drain_adds(bs)

            ids = [pltpu.async_copy(idx_hbm.at[pl.ds(offs[i], ch)],
                                    idxb[js[i]], sems) for i in range(2)]

            def stage_and_add(rows_hbm, mine):
                ds = [pltpu.async_copy(
                    rows_hbm.at[pl.ds(offs[i], ch)], rowsb[js[i]], sems)
                    for i in range(2)]
                ds.append(pltpu.async_copy(
                    prow_hbm.at[pl.ds(offs[mine], ch)], prowb[bs], sems))
                ds.append(pltpu.async_copy(
                    p_hbm.at[pl.ds(offs[mine], ch)], pb[bs], sems))
                for d in ds + ids:
                    d.wait()
                for i in range(2):
                    pltpu.async_copy(rowsb[js[i]], acc.at[idxb[js[i]]],
                                     sema, add=True)
                pltpu.async_copy(pb[bs], accp.at[prowb[bs]], sema, add=True)

            @pl.when(cid == 0)
            def _():
                stage_and_add(ua_hbm, 0)

            @pl.when(cid == 1)
            def _():
                stage_and_add(ub_hbm, 1)

        def body(g, _):
            group(0, 4 * g, g == 0)
            group(2, 4 * g + 2, g == 0)
            return ()

        n_iter = n_ch // 4
        lax.fori_loop(0, n_iter, body, (), unroll=False)
        drain_adds(0)
        drain_adds(2)
        group(0, n_iter * 4, True)
        drain_adds(0)
        plsc.subcore_barrier()

        @pl.when(cid == 0)
        def _():
            pltpu.sync_copy(acc.at[pl.ds(sid * rpt, rpt)],
                            oa_hbm.at[pl.ds(sid * rpt, rpt)])

        @pl.when((cid == 0) & (sid < 10))
        def _():
            pltpu.sync_copy(accp.at[pl.ds(sid * prt, prt)],
                            op_hbm.at[pl.ds(sid * prt, prt)])

        @pl.when(cid == 1)
        def _():
            pltpu.sync_copy(acc.at[pl.ds(sid * rpt, rpt)],
                            ob_hbm.at[pl.ds(sid * rpt, rpt)])

        @pl.when((cid == 1) & (sid < 10))
        def _():
            pltpu.sync_copy(accp.at[pl.ds(sid * prt, prt)],
                            op2_hbm.at[pl.ds(sid * prt, prt)])

    return k(ua, ub, p128, prow, idx, zeros)


def _layer_consts(cp):
    w = cp['W']
    w_top = jnp.concatenate([w[:DIM], jnp.zeros((DIM, HD), jnp.float32)], 0)
    w_bot = w[DIM:]
    att = cp['att'][0]
    a1 = jnp.zeros((HD, 16), jnp.float32)
    a2 = jnp.zeros((HD, 16), jnp.float32)
    for h in range(HEADS):
        a1 = a1.at[h * DIM:(h + 1) * DIM, h].set(att[h, :DIM])
        a2 = a2.at[h * DIM:(h + 1) * DIM, h].set(att[h, DIM:])
    scale = cp['bn_g'] * lax.rsqrt(cp['bn_v'] + 1e-5)
    offs = cp['bn_b'] - cp['bn_m'] * scale
    bns = jnp.zeros((1, 16), jnp.float32).at[0, :HEADS].set(scale)
    bno = jnp.zeros((1, 16), jnp.float32).at[0, :HEADS].set(offs)
    msk = jnp.zeros((1, 16), jnp.float32).at[0, :HEADS].set(1.0)
    r16 = jnp.zeros((16, HD), jnp.float32)
    for h in range(HEADS):
        r16 = r16.at[h, h * DIM:(h + 1) * DIM].set(1.0)
    return [w_top, w_bot, a1, a2, bns, bno, msk, r16]


def _dgn_consts(dp):
    scale = (dp['bn_g'] * lax.rsqrt(dp['bn_v'] + 1e-5)).reshape(GROUPS, DIM)
    offs = (dp['bn_b'].reshape(GROUPS, DIM)
            - dp['bn_m'].reshape(GROUPS, DIM) * scale)
    return dp['lin'], scale, offs.sum(0).reshape(1, DIM)


def _branch(x, edge_index, edge_attr, batch, params, zeros_acc):
    idx_i = edge_index[0].astype(jnp.int32)
    idx_j = edge_index[1].astype(jnp.int32)
    idx_i2d = idx_i.reshape(N_EDGES, 1)
    bvec = batch.astype(jnp.int32).reshape(N_NODES, 1)

    idx_ij = jnp.concatenate([idx_i, idx_j])

    h0 = _pre_mlp(x[:, 1:21], params['preN'], 1000, pad=True)
    e = _pre_mlp(edge_attr, params['preE'], 1000, pad=False)

    h = h0
    for l in range(3):
        econsts = _layer_consts(params['convs'][l])
        lin, sc, off = _dgn_consts(params['dgns'][l])
        nconsts = [params['convs'][l]['bias'].reshape(1, DIM), lin, sc, off]
        xij = _gather_rows(h, idx_ij)
        ua, ub, p128, prow = _edge_kernel(e, xij, idx_i2d, econsts, 1000)
        acca, accb, accp, accp2 = _scatter_add(ua, ub, p128,
                                               prow.reshape(-1),
                                               idx_i, zeros_acc)
        sarr = accp.reshape(NPAD, HEADS)
        sarr2 = accp2.reshape(NPAD, HEADS)
        if l < 2:
            h = _node_kernel(acca, accb, sarr, sarr2, nconsts, 1000)
        else:
            pool = _node_last_kernel(acca, accb, sarr, sarr2, h0, bvec,
                                     nconsts, 1000)
    return pool


@jax.jit
def _forward_impl(wild_x, wild_edge_index, wild_edge_attr, wild_batch,
                  mutant_x, mutant_edge_index, mutant_edge_attr, mutant_batch,
                  params):
    zeros_acc = jnp.zeros((NPAD, UCOLS), jnp.float32)
    pw = _branch(wild_x, wild_edge_index, wild_edge_attr, wild_batch,
                 params, zeros_acc)
    pm = _branch(mutant_x, mutant_edge_index, mutant_edge_attr, mutant_batch,
                 params, zeros_acc)
    out = _final(pw, pm, params['out_W'], params['out_b'])
    return out.reshape(-1)


def kernel(wild_x, wild_edge_index, wild_edge_attr, wild_batch,
           mutant_x, mutant_edge_index, mutant_edge_attr, mutant_batch,
           params):
    return _forward_impl(wild_x, wild_edge_index, wild_edge_attr, wild_batch,
                         mutant_x, mutant_edge_index, mutant_edge_attr,
                         mutant_batch, params)
